# baseline (speedup 1.0000x reference)
import functools

import jax
import jax.numpy as jnp
from jax import lax
from jax.experimental import pallas as pl
from jax.experimental.pallas import tpu as pltpu
from jax.experimental.pallas import tpu_sc as plsc

B = 16384
E = 32
H = 128

NC = 2
NS = 16
NW = NC * NS
BPW = B // NW
LANES = 128
NCH = BPW // LANES

@functools.cache
def _make_sc_gather():
    mesh = plsc.VectorSubcoreMesh(core_axis_name="c", subcore_axis_name="s")

    @functools.partial(
        pl.kernel,
        mesh=mesh,
        out_type=[
            jax.ShapeDtypeStruct((B, E), jnp.float32),
            jax.ShapeDtypeStruct((B, E), jnp.float32),
        ],
        scratch_types=[
            pltpu.VMEM((NCH, LANES), jnp.int32),
            pltpu.VMEM((NCH, LANES), jnp.int32),
            pltpu.VMEM((BPW, E), jnp.float32),
            pltpu.VMEM((BPW, E), jnp.float32),
            pltpu.SemaphoreType.DMA,
        ],
        compiler_params=pltpu.CompilerParams(use_tc_tiling_on_sc=False),
    )
    def _sc_gather(users2d, movies2d, utab, mtab, uout, mout,
                   uidx, midx, urows, mrows, sem):
        wid = lax.axis_index("s") * NC + lax.axis_index("c")
        row0 = wid * NCH
        base = wid * BPW
        pltpu.sync_copy(users2d.at[pl.ds(row0, NCH)], uidx)
        pltpu.sync_copy(movies2d.at[pl.ds(row0, NCH)], midx)
        copies = []
        for j in range(NCH):
            copies.append(pltpu.async_copy(
                utab.at[uidx.at[j]], urows.at[pl.ds(j * LANES, LANES)], sem))
            copies.append(pltpu.async_copy(
                mtab.at[midx.at[j]], mrows.at[pl.ds(j * LANES, LANES)], sem))
        for c in copies:
            c.wait()
        pltpu.sync_copy(urows, uout.at[pl.ds(base, BPW)])
        pltpu.sync_copy(mrows, mout.at[pl.ds(base, BPW)])

    return _sc_gather


BLK = 2048


def _mlp_body(u, m, w1u, w1m, b1, w2, b2, o):
    h = jnp.dot(u[...], w1u[...], preferred_element_type=jnp.float32)
    h = h + jnp.dot(m[...], w1m[...], preferred_element_type=jnp.float32)
    h = jnp.maximum(h + b1[...], 0.0)
    o[...] = jnp.dot(h, w2[...], preferred_element_type=jnp.float32) + b2[...]


def _mlp(u, m, w1u, w1m, b1, w2, b2):
    return pl.pallas_call(
        _mlp_body,
        grid=(B // BLK,),
        in_specs=[
            pl.BlockSpec((BLK, E), lambda i: (i, 0)),
            pl.BlockSpec((BLK, E), lambda i: (i, 0)),
            pl.BlockSpec((E, H), lambda i: (0, 0)),
            pl.BlockSpec((E, H), lambda i: (0, 0)),
            pl.BlockSpec((1, H), lambda i: (0, 0)),
            pl.BlockSpec((H, 1), lambda i: (0, 0)),
            pl.BlockSpec((1, 1), lambda i: (0, 0)),
        ],
        out_specs=pl.BlockSpec((BLK, 1), lambda i: (i, 0)),
        out_shape=jax.ShapeDtypeStruct((B, 1), jnp.float32),
    )(u, m, w1u, w1m, b1, w2, b2)


def kernel(users, movies, user_table, movie_table, W1, b1, W2, b2):
    u2 = users.reshape(B // LANES, LANES)
    m2 = movies.reshape(B // LANES, LANES)
    u_emb, m_emb = _make_sc_gather()(u2, m2, user_table, movie_table)
    return _mlp(u_emb, m_emb, W1[:E], W1[E:], b1.reshape(1, H), W2,
                b2.reshape(1, 1))
